# baseline (speedup 1.0000x reference)
import functools

import jax
import jax.numpy as jnp
from jax import lax
from jax.experimental import pallas as pl
from jax.experimental.pallas import tpu as pltpu
from jax.experimental.pallas import tpu_sc as plsc

_SPLIT = 512000


def _pack_body(a_ref, b_ref, out_ref):
    out_ref[...] = jnp.concatenate(
        [a_ref[...].T, b_ref[...].T], axis=1)


def _pack_table(tab_t):
    embed, n = tab_t.shape
    blk = 256
    grid = (_SPLIT // blk,)
    last_blk = (n - 1) // blk
    return pl.pallas_call(
        _pack_body,
        grid=grid,
        in_specs=[
            pl.BlockSpec((embed, blk), lambda g: (0, g)),
            pl.BlockSpec(
                (embed, blk),
                lambda g: (0, jnp.minimum(g + _SPLIT // blk, last_blk))),
        ],
        out_specs=pl.BlockSpec((blk, 2 * embed), lambda g: (g, 0)),
        out_shape=jax.ShapeDtypeStruct((_SPLIT, 2 * embed), jnp.float32),
    )(tab_t, tab_t)


def _sc_gather(ku, ki, upacked, ipacked):
    info = plsc.get_sparse_core_info()
    nw = info.num_cores * info.num_subcores
    batch = ku.shape[0]
    width = upacked.shape[1]
    b_per_w = batch // nw
    half = b_per_w // 2

    mesh = plsc.VectorSubcoreMesh(core_axis_name="c", subcore_axis_name="s")
    out_struct = jax.ShapeDtypeStruct((batch, width), jnp.float32)

    @functools.partial(
        pl.kernel,
        mesh=mesh,
        compiler_params=pltpu.CompilerParams(needs_layout_passes=False),
        out_type=[out_struct, out_struct],
        scratch_types=[
            pltpu.VMEM((b_per_w,), jnp.int32),
            pltpu.VMEM((b_per_w,), jnp.int32),
            pltpu.VMEM((half, 128), jnp.float32),
            pltpu.VMEM((half, 128), jnp.float32),
            pltpu.SemaphoreType.DMA,
            pltpu.SemaphoreType.DMA,
            pltpu.SemaphoreType.DMA,
            pltpu.SemaphoreType.DMA,
        ],
    )
    def gather_kernel(ku_hbm, ki_hbm, utab_hbm, itab_hbm,
                      outu_hbm, outi_hbm,
                      idx_u, idx_i, buf_u, buf_i,
                      sem_u, sem_i, sem_wu, sem_wi):
        wid = lax.axis_index("s") * info.num_cores + lax.axis_index("c")
        base = wid * b_per_w
        pltpu.sync_copy(ku_hbm.at[pl.ds(base, b_per_w)], idx_u)
        pltpu.sync_copy(ki_hbm.at[pl.ds(base, b_per_w)], idx_i)

        for h in range(2):
            osl = pl.ds(base + h * half, half)
            isl = pl.ds(h * half, half)
            gu = pltpu.async_copy(
                utab_hbm.at[idx_u.at[isl]], buf_u, sem_u)
            gi = pltpu.async_copy(
                itab_hbm.at[idx_i.at[isl]], buf_i, sem_i)
            gu.wait()
            wu = pltpu.async_copy(buf_u, outu_hbm.at[osl], sem_wu)
            gi.wait()
            wi = pltpu.async_copy(buf_i, outi_hbm.at[osl], sem_wi)
            wu.wait()
            wi.wait()

    return gather_kernel(ku, ki, upacked, ipacked)


def _mlp_body(xu_ref, xi_ref, mu_ref, mi_ref, w1u_ref, w1i_ref, b1_ref,
              w2_ref, b2_ref, out_ref):
    embed = w1u_ref.shape[1]
    xu2 = xu_ref[...]
    xi2 = xi_ref[...]
    mu = mu_ref[...]
    mi = mi_ref[...]
    xu = xu2[:, :embed] * (1.0 - mu) + xu2[:, embed:] * mu
    xi = xi2[:, :embed] * (1.0 - mi) + xi2[:, embed:] * mi
    dn = (((1,), (1,)), ((), ()))
    h = lax.dot_general(xu, w1u_ref[...], dn,
                        preferred_element_type=jnp.float32,
                        precision=lax.Precision.HIGHEST)
    h += lax.dot_general(xi, w1i_ref[...], dn,
                         preferred_element_type=jnp.float32,
                         precision=lax.Precision.HIGHEST)
    h = jnp.maximum(h + b1_ref[...], 0.0)
    out = jnp.sum(h * w2_ref[...], axis=1, keepdims=True)
    out_ref[...] = out + b2_ref[0, 0]


def _tc_mlp(xu2, xi2, mu, mi, W1, b1, W2, b2):
    batch, width = xu2.shape
    embed = width // 2
    hidden = W1.shape[0]
    w1u = W1[:, :embed]
    w1i = W1[:, embed:]
    b1r = b1.reshape(1, hidden)
    b2r = b2.reshape(1, 1)
    blk = 2048
    grid = (batch // blk,)
    return pl.pallas_call(
        _mlp_body,
        grid=grid,
        in_specs=[
            pl.BlockSpec((blk, width), lambda i: (i, 0)),
            pl.BlockSpec((blk, width), lambda i: (i, 0)),
            pl.BlockSpec((blk, embed), lambda i: (i, 0)),
            pl.BlockSpec((blk, embed), lambda i: (i, 0)),
            pl.BlockSpec((hidden, embed), lambda i: (0, 0)),
            pl.BlockSpec((hidden, embed), lambda i: (0, 0)),
            pl.BlockSpec((1, hidden), lambda i: (0, 0)),
            pl.BlockSpec((1, hidden), lambda i: (0, 0)),
            pl.BlockSpec((1, 1), lambda i: (0, 0)),
        ],
        out_specs=pl.BlockSpec((blk, 1), lambda i: (i, 0)),
        out_shape=jax.ShapeDtypeStruct((batch, 1), jnp.float32),
    )(xu2, xi2, mu, mi, w1u, w1i, b1r, W2, b2r)


@jax.jit
def kernel(users, items, user_table, item_table, W1, b1, W2, b2):
    batch = users.shape[0]
    embed = user_table.shape[1]
    upacked = _pack_table(user_table.T)
    ipacked = _pack_table(item_table.T)
    ku = jnp.where(users >= _SPLIT, users - _SPLIT, users)
    ki = jnp.where(items >= _SPLIT, items - _SPLIT, items)
    mu = jnp.broadcast_to(
        (users >= _SPLIT).astype(jnp.float32)[:, None], (batch, embed))
    mi = jnp.broadcast_to(
        (items >= _SPLIT).astype(jnp.float32)[:, None], (batch, embed))
    xu2, xi2 = _sc_gather(ku, ki, upacked, ipacked)
    return _tc_mlp(xu2, xi2, mu, mi, W1, b1, W2, b2)
